# baseline (speedup 1.0000x reference)
import jax
import jax.numpy as jnp
from jax import lax
from jax.experimental import pallas as pl
from jax.experimental.pallas import tpu as pltpu
from jax.experimental.pallas import tpu_sc as plsc

VOCAB = 1_000_000
EMBED = 64
BATCH = 4096
HIST = 200

_NC = 2
_NS = 16
_NW = _NC * _NS
_BPW = BATCH // _NW
_GCH = 40
_NG = HIST // _GCH
_NBUF = 2


def _gather_body(w_hbm, data_hbm, out_hbm, idx_v, bufs_v, gsem, osem):
    wid = lax.axis_index("s") * _NC + lax.axis_index("c")
    b0 = wid * _BPW
    pltpu.sync_copy(data_hbm.at[pl.ds(b0, _BPW)], idx_v)

    def fire_gathers(i, s):
        for k in range(_NG):
            pltpu.async_copy(
                w_hbm.at[idx_v.at[i, pl.ds(k * _GCH, _GCH)]],
                bufs_v.at[s, pl.ds(k * _GCH, _GCH)],
                gsem.at[s])

    def wait_gathers(i, s):
        for k in range(_NG):
            pltpu.make_async_copy(
                w_hbm.at[idx_v.at[i, pl.ds(k * _GCH, _GCH)]],
                bufs_v.at[s, pl.ds(k * _GCH, _GCH)],
                gsem.at[s]).wait()

    def fire_copyout(i, s):
        pltpu.async_copy(bufs_v.at[s], out_hbm.at[b0 + i], osem.at[s])

    def wait_copyout(i, s):
        pltpu.make_async_copy(
            bufs_v.at[s], out_hbm.at[b0 + i], osem.at[s]).wait()

    for s in range(_NBUF):
        fire_gathers(s, s)

    n_groups = _BPW // _NBUF

    def group_step(g, carry):
        for s in range(_NBUF):
            i = g * _NBUF + s
            wait_gathers(i, s)
            fire_copyout(i, s)
        for s in range(_NBUF):
            i = g * _NBUF + s
            wait_copyout(i, s)
            fire_gathers(i + _NBUF, s)
        return carry

    lax.fori_loop(0, n_groups - 1, group_step, 0)

    for s in range(_NBUF):
        i = (n_groups - 1) * _NBUF + s
        wait_gathers(i, s)
        fire_copyout(i, s)
    for s in range(_NBUF):
        i = (n_groups - 1) * _NBUF + s
        wait_copyout(i, s)


def kernel(data, W):
    w_pairs = jax.lax.optimization_barrier(W.reshape(VOCAB // 2, 2 * EMBED))
    w_lin = w_pairs.reshape(VOCAB, EMBED)
    mesh = plsc.VectorSubcoreMesh(core_axis_name="c", subcore_axis_name="s")
    return pl.kernel(
        _gather_body,
        mesh=mesh,
        compiler_params=pltpu.CompilerParams(use_tc_tiling_on_sc=False),
        out_type=jax.ShapeDtypeStruct((BATCH, HIST, EMBED), jnp.float32),
        scratch_types=[
            pltpu.VMEM((_BPW, HIST), jnp.int32),
            pltpu.VMEM((_NBUF, HIST, EMBED), jnp.float32),
            pltpu.SemaphoreType.DMA((_NBUF,)),
            pltpu.SemaphoreType.DMA((_NBUF,)),
        ],
    )(w_lin, data)


# speedup vs baseline: 1.0198x; 1.0198x over previous
import jax
import jax.numpy as jnp
from jax import lax
from jax.experimental import pallas as pl
from jax.experimental.pallas import tpu as pltpu
from jax.experimental.pallas import tpu_sc as plsc

VOCAB = 1_000_000
EMBED = 64
BATCH = 4096
HIST = 200

_NC = 2
_NS = 16
_NW = _NC * _NS
_B = BATCH * HIST
_CHUNK = 128
_CHUNKS = _B // (_NW * _CHUNK)
_NBUF = 8
_GROUPS = _CHUNKS // _NBUF


def _gather_body(w_hbm, data_hbm, out_hbm, idx_v, bufs_v, gsem, osem):
    wid = lax.axis_index("s") * _NC + lax.axis_index("c")
    pltpu.sync_copy(data_hbm.at[wid], idx_v)
    row_base = wid * _CHUNKS * _CHUNK

    def fire_gather(c, b):
        pltpu.async_copy(w_hbm.at[idx_v.at[c]], bufs_v.at[b], gsem.at[b])

    def wait_gather(c, b):
        pltpu.make_async_copy(
            w_hbm.at[idx_v.at[c]], bufs_v.at[b], gsem.at[b]).wait()

    def out_slice(c):
        return out_hbm.at[pl.ds(row_base + c * _CHUNK, _CHUNK)]

    def fire_copyout(c, b):
        pltpu.async_copy(bufs_v.at[b], out_slice(c), osem.at[b])

    def wait_copyout(c, b):
        pltpu.make_async_copy(bufs_v.at[b], out_slice(c), osem.at[b]).wait()

    for b in range(_NBUF):
        fire_gather(b, b)

    def group_step(g, carry):
        for b in range(_NBUF):
            c = g * _NBUF + b
            wait_gather(c, b)
            fire_copyout(c, b)
        for b in range(_NBUF):
            c = g * _NBUF + b
            wait_copyout(c, b)
            fire_gather(c + _NBUF, b)
        return carry

    lax.fori_loop(0, _GROUPS - 1, group_step, 0)

    for b in range(_NBUF):
        c = (_GROUPS - 1) * _NBUF + b
        wait_gather(c, b)
        fire_copyout(c, b)
    for b in range(_NBUF):
        c = (_GROUPS - 1) * _NBUF + b
        wait_copyout(c, b)


def kernel(data, W):
    idx = data.reshape(_NW, _CHUNKS, _CHUNK)
    mesh = plsc.VectorSubcoreMesh(core_axis_name="c", subcore_axis_name="s")
    out_flat = pl.kernel(
        _gather_body,
        mesh=mesh,
        compiler_params=pltpu.CompilerParams(use_tc_tiling_on_sc=False),
        out_type=jax.ShapeDtypeStruct((_B, EMBED), jnp.float32),
        scratch_types=[
            pltpu.VMEM((_CHUNKS, _CHUNK), jnp.int32),
            pltpu.VMEM((_NBUF, _CHUNK, EMBED), jnp.float32),
            pltpu.SemaphoreType.DMA((_NBUF,)),
            pltpu.SemaphoreType.DMA((_NBUF,)),
        ],
    )(W, idx)
    return out_flat.reshape(BATCH, HIST, EMBED)
